# Initial kernel scaffold; baseline (speedup 1.0000x reference)
import jax
import jax.numpy as jnp
from jax.experimental import pallas as pl


def kernel(x):
    raise NotImplementedError("write your pallas kernel here")


# speedup vs baseline: 6.4181x; 6.4181x over previous
import functools

import jax
import jax.numpy as jnp
from jax import lax
from jax.experimental import pallas as pl
from jax.experimental.pallas import tpu as pltpu
from jax.experimental.pallas import tpu_sc as plsc

ROWS = 128
COLS = 32768
KEEP = 16384
L = 16
NVEC = COLS // L
LEVELS = ((20, 11, 2048), (9, 11, 2048), (0, 9, 512))

_info = plsc.get_sparse_core_info()
NC = _info.num_cores
NS = _info.num_subcores
NW = NC * NS
RPW = ROWS // NW


def _tec_body(x_hbm, out_hbm, mask_hbm, row_v, mrow_v, hist_v):
    w = lax.axis_index("s") * NC + lax.axis_index("c")
    ones = jnp.ones((L,), jnp.int32)
    zeros_i = jnp.zeros((L,), jnp.int32)

    def do_row(i, carry):
        r = w * RPW + i
        pltpu.sync_copy(x_hbm.at[r], row_v)

        prefix = jnp.int32(0)
        k = jnp.int32(KEEP)
        for (s, nb, width) in LEVELS:
            hs = s + nb

            def zbody(j, _):
                hist_v[pl.ds(j * L, L)] = zeros_i
                return 0

            lax.fori_loop(0, width // L, zbody, 0)

            def hbody(j, c):
                prefix_c = c
                v = row_v[pl.ds(j * L, L)]
                a = lax.bitcast_convert_type(v, jnp.int32) & jnp.int32(0x7FFFFFFF)
                m = (a >> hs) == (prefix_c >> hs)
                idx = (a >> s) & jnp.int32(width - 1)
                plsc.addupdate_scatter(hist_v, [idx], ones, mask=m)
                return prefix_c

            prefix = lax.fori_loop(0, NVEC, hbody, prefix)

            def sbody(j, c):
                tot, b, cb, kk = c
                h = hist_v[pl.ds(j * L, L)]
                cum = plsc.cumsum(h) + tot
                lt = cum < kk
                b = b + jnp.sum(jnp.where(lt, 1, 0))
                cb = jnp.maximum(cb, jnp.max(jnp.where(lt, cum, 0)))
                return (jnp.max(cum), b, cb, kk)

            _, b, cb, _ = lax.fori_loop(
                0, width // L, sbody,
                (jnp.int32(0), jnp.int32(0), jnp.int32(0), k))
            prefix = prefix | (b << s)
            k = k - cb

        def mbody(j, c):
            thr = c
            v = row_v[pl.ds(j * L, L)]
            a = lax.bitcast_convert_type(v, jnp.int32) & jnp.int32(0x7FFFFFFF)
            m = a >= thr
            row_v[pl.ds(j * L, L)] = jnp.where(m, v, jnp.float32(0.0))
            mrow_v[pl.ds(j * L, L)] = jnp.where(m, jnp.float32(1.0),
                                                jnp.float32(0.0))
            return thr

        lax.fori_loop(0, NVEC, mbody, prefix)
        pltpu.sync_copy(row_v, out_hbm.at[r])
        pltpu.sync_copy(mrow_v, mask_hbm.at[r])
        return carry

    lax.fori_loop(0, RPW, do_row, 0)


@jax.jit
def kernel(x):
    mesh = plsc.VectorSubcoreMesh(core_axis_name="c", subcore_axis_name="s")
    f = pl.kernel(
        _tec_body,
        out_type=(
            jax.ShapeDtypeStruct((ROWS, COLS), jnp.float32),
            jax.ShapeDtypeStruct((ROWS, COLS), jnp.float32),
        ),
        mesh=mesh,
        scratch_types=[
            pltpu.VMEM((COLS,), jnp.float32),
            pltpu.VMEM((COLS,), jnp.float32),
            pltpu.VMEM((2048,), jnp.int32),
        ],
        compiler_params=pltpu.CompilerParams(needs_layout_passes=False),
    )
    out, mask = f(x)
    return (out, mask)


# speedup vs baseline: 6.8223x; 1.0630x over previous
import jax
import jax.numpy as jnp
from jax import lax
from jax.experimental import pallas as pl
from jax.experimental.pallas import tpu as pltpu
from jax.experimental.pallas import tpu_sc as plsc

ROWS = 128
COLS = 32768
KEEP = 16384
L = 16
NVEC = COLS // L
LEVELS = ((20, 11, 2048), (9, 11, 2048), (0, 9, 512))
ONE_F32_BITS = 0x3F800000

_info = plsc.get_sparse_core_info()
NC = _info.num_cores
NS = _info.num_subcores
NW = NC * NS
RPW = ROWS // NW
UN = 8


def _level_select(hist_v, width, k):
    def blkscan(j, c):
        tot, jb, cbb = c
        s = jnp.sum(hist_v[pl.ds(j * L, L)])
        tot = tot + s
        below = tot < k
        jb = jnp.where(below, jb + 1, jb)
        cbb = jnp.where(below, tot, cbb)
        return (tot, jb, cbb)

    _, jb, cbb = lax.fori_loop(
        0, width // L, blkscan,
        (jnp.int32(0), jnp.int32(0), jnp.int32(0)), unroll=UN)
    h = hist_v[pl.ds(jb * L, L)]
    cum = plsc.cumsum(h) + cbb
    lt = cum < k
    nb = jnp.sum(jnp.where(lt, 1, 0))
    b = jb * L + nb
    cb = jnp.maximum(cbb, jnp.max(jnp.where(lt, cum, 0)))
    return b, cb


def _tec_body(x_hbm, out_hbm, mask_hbm, row_v, abs_v, mrow_v, hist_v):
    w = lax.axis_index("s") * NC + lax.axis_index("c")
    ones = jnp.ones((L,), jnp.int32)
    zeros_i = jnp.zeros((L,), jnp.int32)

    def do_row(i, carry):
        r = w * RPW + i
        pltpu.sync_copy(x_hbm.at[r], row_v)

        def z1(j, _):
            hist_v[pl.ds(j * L, L)] = zeros_i
            return 0

        lax.fori_loop(0, 2048 // L, z1, 0, unroll=UN)

        def h1(j, _):
            a = row_v[pl.ds(j * L, L)] & jnp.int32(0x7FFFFFFF)
            abs_v[pl.ds(j * L, L)] = a
            plsc.addupdate_scatter(hist_v, [a >> 20], ones)
            return 0

        lax.fori_loop(0, NVEC, h1, 0, unroll=UN)
        b1, cb1 = _level_select(hist_v, 2048, jnp.int32(KEEP))
        k2 = jnp.int32(KEEP) - cb1

        def z2(j, _):
            hist_v[pl.ds(j * L, L)] = zeros_i
            return 0

        lax.fori_loop(0, 2048 // L, z2, 0, unroll=UN)

        def h2(j, c):
            b1c = c
            a = abs_v[pl.ds(j * L, L)]
            m = (a >> 20) == b1c
            plsc.addupdate_scatter(
                hist_v, [(a >> 9) & jnp.int32(0x7FF)], ones, mask=m)
            return b1c

        lax.fori_loop(0, NVEC, h2, b1, unroll=UN)
        b2, cb2 = _level_select(hist_v, 2048, k2)
        k3 = k2 - cb2
        pref2 = (b1 << 11) | b2

        def z3(j, _):
            hist_v[pl.ds(j * L, L)] = zeros_i
            return 0

        lax.fori_loop(0, 512 // L, z3, 0, unroll=UN)

        def h3(j, c):
            p2 = c
            a = abs_v[pl.ds(j * L, L)]
            m = (a >> 9) == p2
            plsc.addupdate_scatter(hist_v, [a & jnp.int32(0x1FF)], ones,
                                   mask=m)
            return p2

        lax.fori_loop(0, NVEC, h3, pref2, unroll=UN)
        b3, _ = _level_select(hist_v, 512, k3)
        thr = (pref2 << 9) | b3

        def mb(j, c):
            t = c
            a = abs_v[pl.ds(j * L, L)]
            m = a >= t
            row_v[pl.ds(j * L, L)] = jnp.where(m, row_v[pl.ds(j * L, L)],
                                               jnp.int32(0))
            mrow_v[pl.ds(j * L, L)] = jnp.where(m, jnp.int32(ONE_F32_BITS),
                                                jnp.int32(0))
            return t

        lax.fori_loop(0, NVEC, mb, thr, unroll=UN)
        pltpu.sync_copy(row_v, out_hbm.at[r])
        pltpu.sync_copy(mrow_v, mask_hbm.at[r])
        return carry

    lax.fori_loop(0, RPW, do_row, 0)


@jax.jit
def kernel(x):
    xb = lax.bitcast_convert_type(x, jnp.int32)
    mesh = plsc.VectorSubcoreMesh(core_axis_name="c", subcore_axis_name="s")
    f = pl.kernel(
        _tec_body,
        out_type=(
            jax.ShapeDtypeStruct((ROWS, COLS), jnp.int32),
            jax.ShapeDtypeStruct((ROWS, COLS), jnp.int32),
        ),
        mesh=mesh,
        scratch_types=[
            pltpu.VMEM((COLS,), jnp.int32),
            pltpu.VMEM((COLS,), jnp.int32),
            pltpu.VMEM((COLS,), jnp.int32),
            pltpu.VMEM((2048,), jnp.int32),
        ],
        compiler_params=pltpu.CompilerParams(needs_layout_passes=False),
    )
    out_b, mask_b = f(xb)
    return (lax.bitcast_convert_type(out_b, jnp.float32),
            lax.bitcast_convert_type(mask_b, jnp.float32))


# speedup vs baseline: 18.0388x; 2.6441x over previous
import jax
import jax.numpy as jnp
from jax import lax
from jax.experimental import pallas as pl
from jax.experimental.pallas import tpu as pltpu
from jax.experimental.pallas import tpu_sc as plsc

ROWS = 128
COLS = 32768
KEEP = 16384
L = 16
NVEC = COLS // L
LEVELS = ((20, 11, 2048), (9, 11, 2048), (0, 9, 512))
ONE_F32_BITS = 0x3F800000

_info = plsc.get_sparse_core_info()
NC = _info.num_cores
NS = _info.num_subcores
NW = NC * NS
RPW = ROWS // NW
UN = 8


def _level_select(hist_v, width, k):
    def blkscan(j, c):
        tot, jb, cbb = c
        s = jnp.sum(hist_v[pl.ds(j * L, L)])
        tot = tot + s
        below = tot < k
        jb = jnp.where(below, jb + 1, jb)
        cbb = jnp.where(below, tot, cbb)
        return (tot, jb, cbb)

    _, jb, cbb = lax.fori_loop(
        0, width // L, blkscan,
        (jnp.int32(0), jnp.int32(0), jnp.int32(0)), unroll=UN)
    h = hist_v[pl.ds(jb * L, L)]
    cum = plsc.cumsum(h) + cbb
    lt = cum < k
    nb = jnp.sum(jnp.where(lt, 1, 0))
    b = jb * L + nb
    cb = jnp.maximum(cbb, jnp.max(jnp.where(lt, cum, 0)))
    return b, cb


def _tec_body(x_hbm, out_hbm, mask_hbm, row_v, abs_v, mrow_v, hist_v):
    w = lax.axis_index("s") * NC + lax.axis_index("c")
    ones = jnp.ones((L,), jnp.int32)
    zeros_i = jnp.zeros((L,), jnp.int32)

    def do_row(i, carry):
        r = w * RPW + i
        pltpu.sync_copy(x_hbm.at[r], row_v)

        @plsc.parallel_loop(0, 2048, L, unroll=UN)
        def _(j):
            hist_v[pl.ds(j, L)] = zeros_i

        @plsc.parallel_loop(0, COLS, L, unroll=UN)
        def _(j):
            a = row_v[pl.ds(j, L)] & jnp.int32(0x7FFFFFFF)
            abs_v[pl.ds(j, L)] = a
            plsc.addupdate_scatter(hist_v, [a >> 20], ones)

        b1, cb1 = _level_select(hist_v, 2048, jnp.int32(KEEP))
        k2 = jnp.int32(KEEP) - cb1

        @plsc.parallel_loop(0, 2048, L, unroll=UN)
        def _(j):
            hist_v[pl.ds(j, L)] = zeros_i

        @plsc.parallel_loop(0, COLS, L, unroll=UN)
        def _(j):
            a = abs_v[pl.ds(j, L)]
            m = (a >> 20) == b1
            plsc.addupdate_scatter(
                hist_v, [(a >> 9) & jnp.int32(0x7FF)], ones, mask=m)

        b2, cb2 = _level_select(hist_v, 2048, k2)
        k3 = k2 - cb2
        pref2 = (b1 << 11) | b2

        @plsc.parallel_loop(0, 512, L, unroll=UN)
        def _(j):
            hist_v[pl.ds(j, L)] = zeros_i

        @plsc.parallel_loop(0, COLS, L, unroll=UN)
        def _(j):
            a = abs_v[pl.ds(j, L)]
            m = (a >> 9) == pref2
            plsc.addupdate_scatter(hist_v, [a & jnp.int32(0x1FF)], ones,
                                   mask=m)

        b3, _ = _level_select(hist_v, 512, k3)
        thr = (pref2 << 9) | b3

        @plsc.parallel_loop(0, COLS, L, unroll=UN)
        def _(j):
            a = abs_v[pl.ds(j, L)]
            m = a >= thr
            row_v[pl.ds(j, L)] = jnp.where(m, row_v[pl.ds(j, L)],
                                           jnp.int32(0))
            mrow_v[pl.ds(j, L)] = jnp.where(m, jnp.int32(ONE_F32_BITS),
                                            jnp.int32(0))
        pltpu.sync_copy(row_v, out_hbm.at[r])
        pltpu.sync_copy(mrow_v, mask_hbm.at[r])
        return carry

    lax.fori_loop(0, RPW, do_row, 0)


@jax.jit
def kernel(x):
    xb = lax.bitcast_convert_type(x, jnp.int32)
    mesh = plsc.VectorSubcoreMesh(core_axis_name="c", subcore_axis_name="s")
    f = pl.kernel(
        _tec_body,
        out_type=(
            jax.ShapeDtypeStruct((ROWS, COLS), jnp.int32),
            jax.ShapeDtypeStruct((ROWS, COLS), jnp.int32),
        ),
        mesh=mesh,
        scratch_types=[
            pltpu.VMEM((COLS,), jnp.int32),
            pltpu.VMEM((COLS,), jnp.int32),
            pltpu.VMEM((COLS,), jnp.int32),
            pltpu.VMEM((2048,), jnp.int32),
        ],
        compiler_params=pltpu.CompilerParams(needs_layout_passes=False),
    )
    out_b, mask_b = f(xb)
    return (lax.bitcast_convert_type(out_b, jnp.float32),
            lax.bitcast_convert_type(mask_b, jnp.float32))


# speedup vs baseline: 20.2581x; 1.1230x over previous
import jax
import jax.numpy as jnp
from jax import lax
from jax.experimental import pallas as pl
from jax.experimental.pallas import tpu as pltpu
from jax.experimental.pallas import tpu_sc as plsc

ROWS = 128
COLS = 32768
KEEP = 16384
L = 16
ONE_F32_BITS = 0x3F800000

_info = plsc.get_sparse_core_info()
NC = _info.num_cores
NS = _info.num_subcores
NW = NC * NS
RPW = ROWS // NW
UN = 8


def _level_select(hist_v, width, k):
    @plsc.parallel_loop(0, width, L, unroll=UN,
                        carry=(jnp.int32(0), jnp.int32(0), jnp.int32(0)))
    def blk(j, c):
        tot, jb, cbb = c
        s = jnp.sum(hist_v[pl.ds(j, L)])
        tot = tot + s
        below = tot < k
        jb = jnp.where(below, jb + 1, jb)
        cbb = jnp.where(below, tot, cbb)
        return (tot, jb, cbb)

    _, jb, cbb = blk
    h = hist_v[pl.ds(jb * L, L)]
    cum = plsc.cumsum(h) + cbb
    lt = cum < k
    nb = jnp.sum(jnp.where(lt, 1, 0))
    b = jb * L + nb
    cb = jnp.maximum(cbb, jnp.max(jnp.where(lt, cum, 0)))
    return b, cb


def _find_threshold(buf, hist_v, ones, zeros_i):
    @plsc.parallel_loop(0, 2048, L, unroll=UN)
    def _(j):
        hist_v[pl.ds(j, L)] = zeros_i

    @plsc.parallel_loop(0, COLS, L, unroll=UN)
    def _(j):
        a = buf[pl.ds(j, L)] & jnp.int32(0x7FFFFFFF)
        plsc.addupdate_scatter(hist_v, [a >> 20], ones)

    b1, cb1 = _level_select(hist_v, 2048, jnp.int32(KEEP))
    k2 = jnp.int32(KEEP) - cb1

    @plsc.parallel_loop(0, 2048, L, unroll=UN)
    def _(j):
        hist_v[pl.ds(j, L)] = zeros_i

    @plsc.parallel_loop(0, COLS, L, unroll=UN)
    def _(j):
        a = buf[pl.ds(j, L)] & jnp.int32(0x7FFFFFFF)
        m = (a >> 20) == b1
        plsc.addupdate_scatter(
            hist_v, [(a >> 9) & jnp.int32(0x7FF)], ones, mask=m)

    b2, cb2 = _level_select(hist_v, 2048, k2)
    k3 = k2 - cb2
    pref2 = (b1 << 11) | b2

    @plsc.parallel_loop(0, 512, L, unroll=UN)
    def _(j):
        hist_v[pl.ds(j, L)] = zeros_i

    @plsc.parallel_loop(0, COLS, L, unroll=UN)
    def _(j):
        a = buf[pl.ds(j, L)] & jnp.int32(0x7FFFFFFF)
        m = (a >> 9) == pref2
        plsc.addupdate_scatter(hist_v, [a & jnp.int32(0x1FF)], ones, mask=m)

    b3, _ = _level_select(hist_v, 512, k3)
    return (pref2 << 9) | b3


def _tec_body(x_hbm, out_hbm, mask_hbm, buf0, buf1, mrow_v, hist_v,
              in_sem0, in_sem1, out_sem0, out_sem1, mask_sem):
    w = lax.axis_index("s") * NC + lax.axis_index("c")
    ones = jnp.ones((L,), jnp.int32)
    zeros_i = jnp.zeros((L,), jnp.int32)
    bufs = (buf0, buf1)
    in_sems = (in_sem0, in_sem1)
    out_sems = (out_sem0, out_sem1)
    r0 = w * RPW

    pltpu.make_async_copy(x_hbm.at[r0], buf0, in_sem0).start()
    for i in range(RPW):
        buf = bufs[i % 2]
        other = bufs[(i + 1) % 2]
        r = r0 + i
        pltpu.make_async_copy(x_hbm.at[r], buf, in_sems[i % 2]).wait()

        thr = _find_threshold(buf, hist_v, ones, zeros_i)

        if i + 1 < RPW:
            if i >= 1:
                pltpu.make_async_copy(other, out_hbm.at[r - 1],
                                      out_sems[(i + 1) % 2]).wait()
            pltpu.make_async_copy(x_hbm.at[r + 1], other,
                                  in_sems[(i + 1) % 2]).start()
        if i >= 1:
            pltpu.make_async_copy(mrow_v, mask_hbm.at[r - 1], mask_sem).wait()

        @plsc.parallel_loop(0, COLS, L, unroll=UN)
        def _(j):
            a = buf[pl.ds(j, L)] & jnp.int32(0x7FFFFFFF)
            m = a >= thr
            buf[pl.ds(j, L)] = jnp.where(m, buf[pl.ds(j, L)], jnp.int32(0))
            mrow_v[pl.ds(j, L)] = jnp.where(m, jnp.int32(ONE_F32_BITS),
                                            jnp.int32(0))

        pltpu.make_async_copy(buf, out_hbm.at[r], out_sems[i % 2]).start()
        pltpu.make_async_copy(mrow_v, mask_hbm.at[r], mask_sem).start()

    pltpu.make_async_copy(bufs[(RPW - 1) % 2], out_hbm.at[r0 + RPW - 1],
                          out_sems[(RPW - 1) % 2]).wait()
    pltpu.make_async_copy(bufs[RPW % 2], out_hbm.at[r0 + RPW - 2],
                          out_sems[RPW % 2]).wait()
    pltpu.make_async_copy(mrow_v, mask_hbm.at[r0 + RPW - 1], mask_sem).wait()


@jax.jit
def kernel(x):
    xb = lax.bitcast_convert_type(x, jnp.int32)
    mesh = plsc.VectorSubcoreMesh(core_axis_name="c", subcore_axis_name="s")
    f = pl.kernel(
        _tec_body,
        out_type=(
            jax.ShapeDtypeStruct((ROWS, COLS), jnp.int32),
            jax.ShapeDtypeStruct((ROWS, COLS), jnp.int32),
        ),
        mesh=mesh,
        scratch_types=[
            pltpu.VMEM((COLS,), jnp.int32),
            pltpu.VMEM((COLS,), jnp.int32),
            pltpu.VMEM((COLS,), jnp.int32),
            pltpu.VMEM((2048,), jnp.int32),
            pltpu.SemaphoreType.DMA,
            pltpu.SemaphoreType.DMA,
            pltpu.SemaphoreType.DMA,
            pltpu.SemaphoreType.DMA,
            pltpu.SemaphoreType.DMA,
        ],
        compiler_params=pltpu.CompilerParams(needs_layout_passes=False),
    )
    out_b, mask_b = f(xb)
    return (lax.bitcast_convert_type(out_b, jnp.float32),
            lax.bitcast_convert_type(mask_b, jnp.float32))
